# Initial kernel scaffold; baseline (speedup 1.0000x reference)
import jax
import jax.numpy as jnp
from jax.experimental import pallas as pl


def kernel(x, a, i, w0_1, w1_1, b1, p, w0_2, w1_2, b2, w0_3, w1_3, b3, wd, bd):
    raise NotImplementedError("write your pallas kernel here")


# speedup vs baseline: 2.9686x; 2.9686x over previous
import functools

import jax
import jax.numpy as jnp
import numpy as np
from jax.experimental import pallas as pl

N = 10000
BR = 200


def _mm_body(a_ref, v_ref, o_ref):
    o_ref[...] = jnp.dot(a_ref[...], v_ref[...],
                         preferred_element_type=jnp.float32)


def _big_mm(a, v):
    m = v.shape[1]
    return pl.pallas_call(
        _mm_body,
        grid=(N // BR,),
        in_specs=[
            pl.BlockSpec((BR, N), lambda i: (i, 0)),
            pl.BlockSpec((N, m), lambda i: (0, 0)),
        ],
        out_specs=pl.BlockSpec((BR, m), lambda i: (i, 0)),
        out_shape=jax.ShapeDtypeStruct((N, m), jnp.float32),
    )(a, v)


def _conv_body(a_ref, v_ref, dinv_ref, xb_ref, p_ref, rm_ref, o_feat, o_y, o_cs):
    z = jnp.dot(a_ref[...], v_ref[...], preferred_element_type=jnp.float32)
    feat = jax.nn.relu(dinv_ref[...] * z + xb_ref[...]) * rm_ref[...]
    o_feat[...] = feat
    o_y[...] = jnp.dot(feat, p_ref[...], preferred_element_type=jnp.float32)
    o_cs[...] = jnp.sum(feat, axis=0, keepdims=True)[None]


def _conv_pass(a, v, dinv, xb, p_col, rm):
    h = v.shape[1]
    return pl.pallas_call(
        _conv_body,
        grid=(N // BR,),
        in_specs=[
            pl.BlockSpec((BR, N), lambda i: (i, 0)),
            pl.BlockSpec((N, h), lambda i: (0, 0)),
            pl.BlockSpec((BR, 1), lambda i: (i, 0)),
            pl.BlockSpec((BR, h), lambda i: (i, 0)),
            pl.BlockSpec((h, 1), lambda i: (0, 0)),
            pl.BlockSpec((BR, 1), lambda i: (i, 0)),
        ],
        out_specs=[
            pl.BlockSpec((BR, h), lambda i: (i, 0)),
            pl.BlockSpec((BR, 1), lambda i: (i, 0)),
            pl.BlockSpec((1, 1, h), lambda i: (i, 0, 0)),
        ],
        out_shape=[
            jax.ShapeDtypeStruct((N, h), jnp.float32),
            jax.ShapeDtypeStruct((N, 1), jnp.float32),
            jax.ShapeDtypeStruct((N // BR, 1, h), jnp.float32),
        ],
    )(a, v, dinv, xb, p_col, rm)


def _small_mm_body(x_ref, w_ref, b_ref, o_ref):
    o_ref[...] = (jnp.dot(x_ref[...], w_ref[...],
                          preferred_element_type=jnp.float32) + b_ref[...])


def _small_mm(x, w, b):
    n, f = x.shape
    h = w.shape[1]
    return pl.pallas_call(
        _small_mm_body,
        in_specs=[pl.BlockSpec((n, f), lambda: (0, 0)),
                  pl.BlockSpec((f, h), lambda: (0, 0)),
                  pl.BlockSpec((1, h), lambda: (0, 0))],
        out_specs=pl.BlockSpec((n, h), lambda: (0, 0)),
        out_shape=jax.ShapeDtypeStruct((n, h), jnp.float32),
    )(x, w, b)


def _dinv(deg):
    return jnp.where(deg > 0, jax.lax.rsqrt(deg), 0.0)


def _topk_mask(y, k):
    _, idx = jax.lax.top_k(y, k)
    return jnp.zeros((N,), jnp.float32).at[idx].set(1.0)


def kernel(x, a, i, w0_1, w1_1, b1, p, w0_2, w1_2, b2, w0_3, w1_3, b3, wd, bd):
    del i
    pn = (p / jnp.sqrt(jnp.sum(p * p)))[:, None]
    neg = jnp.float32(-jnp.inf)

    deg0 = _big_mm(a, jnp.ones((N, 1), jnp.float32))
    di0 = _dinv(deg0)
    xw0 = _small_mm(x, w0_1, jnp.zeros((1, 32), jnp.float32))
    xb0 = _small_mm(x, w1_1, b1[None, :])
    ones_col = jnp.ones((N, 1), jnp.float32)
    x1f, y1, _ = _conv_pass(a, di0 * xw0, di0, xb0, pn, ones_col)
    y1 = y1[:, 0]

    k1 = int(np.ceil(0.5 * N))
    m1 = _topk_mask(y1, k1)
    g1 = jax.nn.sigmoid(y1) * m1
    x1g = x1f * g1[:, None]

    deg1 = _big_mm(a, m1[:, None])
    di1 = _dinv(deg1)
    xw1 = _small_mm(x1g, w0_2, jnp.zeros((1, 32), jnp.float32))
    xb1 = _small_mm(x1g, w1_2, b2[None, :])
    x2f, y2, _ = _conv_pass(a, m1[:, None] * di1 * xw1, di1, xb1, pn,
                            m1[:, None])
    y2 = jnp.where(m1 > 0, y2[:, 0], neg)

    k2 = int(np.ceil(0.5 * k1))
    m2 = _topk_mask(y2, k2)
    g2 = jax.nn.sigmoid(y2) * m2
    x2g = x2f * g2[:, None]

    deg2 = _big_mm(a, m2[:, None])
    di2 = _dinv(deg2)
    xw2 = _small_mm(x2g, w0_3, jnp.zeros((1, 32), jnp.float32))
    xb2 = _small_mm(x2g, w1_3, b3[None, :])
    _, _, cs = _conv_pass(a, m2[:, None] * di2 * xw2, di2, xb2, pn,
                          m2[:, None])

    pooled = jnp.sum(cs, axis=0) / k2
    return pooled @ wd + bd[None, :]


# speedup vs baseline: 3.2895x; 1.1081x over previous
import functools

import jax
import jax.numpy as jnp
import numpy as np
from jax.experimental import pallas as pl

N = 10000
BR = 200


def _cast_deg_body(a_ref, a16_ref, deg_ref):
    blk = a_ref[...]
    a16_ref[...] = blk.astype(jnp.bfloat16)
    deg_ref[...] = jnp.sum(blk, axis=1, keepdims=True)


def _cast_deg(a):
    return pl.pallas_call(
        _cast_deg_body,
        grid=(N // BR,),
        in_specs=[pl.BlockSpec((BR, N), lambda i: (i, 0))],
        out_specs=[pl.BlockSpec((BR, N), lambda i: (i, 0)),
                   pl.BlockSpec((BR, 1), lambda i: (i, 0))],
        out_shape=[jax.ShapeDtypeStruct((N, N), jnp.bfloat16),
                   jax.ShapeDtypeStruct((N, 1), jnp.float32)],
    )(a)


def _mm_body(a_ref, v_ref, o_ref):
    o_ref[...] = jnp.dot(a_ref[...], v_ref[...],
                         preferred_element_type=jnp.float32)


def _big_mm(a, v):
    m = v.shape[1]
    return pl.pallas_call(
        _mm_body,
        grid=(N // BR,),
        in_specs=[
            pl.BlockSpec((BR, N), lambda i: (i, 0)),
            pl.BlockSpec((N, m), lambda i: (0, 0)),
        ],
        out_specs=pl.BlockSpec((BR, m), lambda i: (i, 0)),
        out_shape=jax.ShapeDtypeStruct((N, m), jnp.float32),
    )(a, v.astype(a.dtype))


def _conv_body(a_ref, v_ref, dinv_ref, xb_ref, p_ref, rm_ref, o_feat, o_y, o_cs):
    z = jnp.dot(a_ref[...], v_ref[...], preferred_element_type=jnp.float32)
    feat = jax.nn.relu(dinv_ref[...] * z + xb_ref[...]) * rm_ref[...]
    o_feat[...] = feat
    o_y[...] = jnp.dot(feat, p_ref[...], preferred_element_type=jnp.float32)
    o_cs[...] = jnp.sum(feat, axis=0, keepdims=True)[None]


def _conv_pass(a, v, dinv, xb, p_col, rm):
    h = v.shape[1]
    return pl.pallas_call(
        _conv_body,
        grid=(N // BR,),
        in_specs=[
            pl.BlockSpec((BR, N), lambda i: (i, 0)),
            pl.BlockSpec((N, h), lambda i: (0, 0)),
            pl.BlockSpec((BR, 1), lambda i: (i, 0)),
            pl.BlockSpec((BR, h), lambda i: (i, 0)),
            pl.BlockSpec((h, 1), lambda i: (0, 0)),
            pl.BlockSpec((BR, 1), lambda i: (i, 0)),
        ],
        out_specs=[
            pl.BlockSpec((BR, h), lambda i: (i, 0)),
            pl.BlockSpec((BR, 1), lambda i: (i, 0)),
            pl.BlockSpec((1, 1, h), lambda i: (i, 0, 0)),
        ],
        out_shape=[
            jax.ShapeDtypeStruct((N, h), jnp.float32),
            jax.ShapeDtypeStruct((N, 1), jnp.float32),
            jax.ShapeDtypeStruct((N // BR, 1, h), jnp.float32),
        ],
    )(a, v.astype(a.dtype), dinv, xb, p_col, rm)


def _small_mm_body(x_ref, w_ref, b_ref, o_ref):
    o_ref[...] = (jnp.dot(x_ref[...], w_ref[...],
                          preferred_element_type=jnp.float32) + b_ref[...])


def _small_mm(x, w, b):
    n, f = x.shape
    h = w.shape[1]
    return pl.pallas_call(
        _small_mm_body,
        in_specs=[pl.BlockSpec((n, f), lambda: (0, 0)),
                  pl.BlockSpec((f, h), lambda: (0, 0)),
                  pl.BlockSpec((1, h), lambda: (0, 0))],
        out_specs=pl.BlockSpec((n, h), lambda: (0, 0)),
        out_shape=jax.ShapeDtypeStruct((n, h), jnp.float32),
    )(x, w, b)


def _dinv(deg):
    return jnp.where(deg > 0, jax.lax.rsqrt(deg), 0.0)


def _topk_mask(y, k):
    _, idx = jax.lax.top_k(y, k)
    return jnp.zeros((N,), jnp.float32).at[idx].set(1.0)


def kernel(x, a, i, w0_1, w1_1, b1, p, w0_2, w1_2, b2, w0_3, w1_3, b3, wd, bd):
    del i
    pn = (p / jnp.sqrt(jnp.sum(p * p)))[:, None]
    neg = jnp.float32(-jnp.inf)

    a16, deg0 = _cast_deg(a)
    di0 = _dinv(deg0)
    xw0 = _small_mm(x, w0_1, jnp.zeros((1, 32), jnp.float32))
    xb0 = _small_mm(x, w1_1, b1[None, :])
    ones_col = jnp.ones((N, 1), jnp.float32)
    x1f, y1, _ = _conv_pass(a16, di0 * xw0, di0, xb0, pn, ones_col)
    y1 = y1[:, 0]

    k1 = int(np.ceil(0.5 * N))
    m1 = _topk_mask(y1, k1)
    g1 = jax.nn.sigmoid(y1) * m1
    x1g = x1f * g1[:, None]

    deg1 = _big_mm(a16, m1[:, None])
    di1 = _dinv(deg1)
    xw1 = _small_mm(x1g, w0_2, jnp.zeros((1, 32), jnp.float32))
    xb1 = _small_mm(x1g, w1_2, b2[None, :])
    x2f, y2, _ = _conv_pass(a16, m1[:, None] * di1 * xw1, di1, xb1, pn,
                            m1[:, None])
    y2 = jnp.where(m1 > 0, y2[:, 0], neg)

    k2 = int(np.ceil(0.5 * k1))
    m2 = _topk_mask(y2, k2)
    g2 = jax.nn.sigmoid(y2) * m2
    x2g = x2f * g2[:, None]

    deg2 = _big_mm(a16, m2[:, None])
    di2 = _dinv(deg2)
    xw2 = _small_mm(x2g, w0_3, jnp.zeros((1, 32), jnp.float32))
    xb2 = _small_mm(x2g, w1_3, b3[None, :])
    _, _, cs = _conv_pass(a16, m2[:, None] * di2 * xw2, di2, xb2, pn,
                          m2[:, None])

    pooled = jnp.sum(cs, axis=0) / k2
    return pooled @ wd + bd[None, :]


# speedup vs baseline: 3.4563x; 1.0507x over previous
import functools

import jax
import jax.numpy as jnp
import numpy as np
from jax.experimental import pallas as pl

N = 10000
BR = 200


def _cast_deg_body(a_ref, a16_ref, deg_ref):
    blk = a_ref[...]
    a16_ref[...] = blk.astype(jnp.bfloat16)
    deg_ref[...] = jnp.sum(blk, axis=1, keepdims=True)


def _cast_deg(a):
    return pl.pallas_call(
        _cast_deg_body,
        grid=(N // BR,),
        in_specs=[pl.BlockSpec((BR, N), lambda i: (i, 0))],
        out_specs=[pl.BlockSpec((BR, N), lambda i: (i, 0)),
                   pl.BlockSpec((BR, 1), lambda i: (i, 0))],
        out_shape=[jax.ShapeDtypeStruct((N, N), jnp.bfloat16),
                   jax.ShapeDtypeStruct((N, 1), jnp.float32)],
    )(a)


def _mm_body(a_ref, v_ref, o_ref):
    o_ref[...] = jnp.dot(a_ref[...], v_ref[...],
                         preferred_element_type=jnp.float32)


def _big_mm(a, v):
    m = v.shape[1]
    return pl.pallas_call(
        _mm_body,
        grid=(N // BR,),
        in_specs=[
            pl.BlockSpec((BR, N), lambda i: (i, 0)),
            pl.BlockSpec((N, m), lambda i: (0, 0)),
        ],
        out_specs=pl.BlockSpec((BR, m), lambda i: (i, 0)),
        out_shape=jax.ShapeDtypeStruct((N, m), jnp.float32),
    )(a, v.astype(a.dtype))


def _conv_body(a_ref, v_ref, dinv_ref, xb_ref, p_ref, rm_ref, o_feat, o_y, o_cs):
    z = jnp.dot(a_ref[...], v_ref[...], preferred_element_type=jnp.float32)
    feat = jax.nn.relu(dinv_ref[...] * z + xb_ref[...]) * rm_ref[...]
    o_feat[...] = feat
    o_y[...] = jnp.dot(feat, p_ref[...], preferred_element_type=jnp.float32)
    o_cs[...] = jnp.sum(feat, axis=0, keepdims=True)[None]


def _conv_pass(a, v, dinv, xb, p_col, rm):
    h = v.shape[1]
    return pl.pallas_call(
        _conv_body,
        grid=(N // BR,),
        in_specs=[
            pl.BlockSpec((BR, N), lambda i: (i, 0)),
            pl.BlockSpec((N, h), lambda i: (0, 0)),
            pl.BlockSpec((BR, 1), lambda i: (i, 0)),
            pl.BlockSpec((BR, h), lambda i: (i, 0)),
            pl.BlockSpec((h, 1), lambda i: (0, 0)),
            pl.BlockSpec((BR, 1), lambda i: (i, 0)),
        ],
        out_specs=[
            pl.BlockSpec((BR, h), lambda i: (i, 0)),
            pl.BlockSpec((BR, 1), lambda i: (i, 0)),
            pl.BlockSpec((1, 1, h), lambda i: (i, 0, 0)),
        ],
        out_shape=[
            jax.ShapeDtypeStruct((N, h), jnp.float32),
            jax.ShapeDtypeStruct((N, 1), jnp.float32),
            jax.ShapeDtypeStruct((N // BR, 1, h), jnp.float32),
        ],
    )(a, v.astype(a.dtype), dinv, xb, p_col, rm)


def _small_mm_body(x_ref, w_ref, b_ref, o_ref):
    o_ref[...] = (jnp.dot(x_ref[...], w_ref[...],
                          preferred_element_type=jnp.float32) + b_ref[...])


def _small_mm(x, w, b):
    n, f = x.shape
    h = w.shape[1]
    return pl.pallas_call(
        _small_mm_body,
        in_specs=[pl.BlockSpec((n, f), lambda: (0, 0)),
                  pl.BlockSpec((f, h), lambda: (0, 0)),
                  pl.BlockSpec((1, h), lambda: (0, 0))],
        out_specs=pl.BlockSpec((n, h), lambda: (0, 0)),
        out_shape=jax.ShapeDtypeStruct((n, h), jnp.float32),
    )(x, w, b)


def _dinv(deg):
    return jnp.where(deg > 0, jax.lax.rsqrt(deg), 0.0)


NP_ROWS = 80
NP_COLS = 128


def _thresh_body(k, y_ref, valid_ref, o_ref):
    yi = jax.lax.bitcast_convert_type(y_ref[...], jnp.int32)
    key = yi ^ jnp.where(yi < 0, jnp.int32(0x7FFFFFFF), jnp.int32(0))
    u = jax.lax.bitcast_convert_type(key ^ jnp.int32(-0x80000000), jnp.uint32)
    u = jnp.where(valid_ref[...] > 0, u, jnp.uint32(0))

    def body(b, t):
        cand = t | (jnp.uint32(1) << (jnp.uint32(31) - b.astype(jnp.uint32)))
        cnt = jnp.sum((u >= cand).astype(jnp.int32))
        return jnp.where(cnt >= k, cand, t)

    t = jax.lax.fori_loop(0, 32, body, jnp.uint32(0))
    o_ref[...] = (u >= t).astype(jnp.float32)


def _topk_mask(y, k, valid):
    pad = NP_ROWS * NP_COLS - N
    y2 = jnp.pad(y, (0, pad), constant_values=-jnp.inf).reshape(NP_ROWS, NP_COLS)
    v2 = jnp.pad(valid, (0, pad)).reshape(NP_ROWS, NP_COLS)
    m = pl.pallas_call(
        functools.partial(_thresh_body, k),
        in_specs=[pl.BlockSpec((NP_ROWS, NP_COLS), lambda: (0, 0)),
                  pl.BlockSpec((NP_ROWS, NP_COLS), lambda: (0, 0))],
        out_specs=pl.BlockSpec((NP_ROWS, NP_COLS), lambda: (0, 0)),
        out_shape=jax.ShapeDtypeStruct((NP_ROWS, NP_COLS), jnp.float32),
    )(y2, v2)
    return m.reshape(-1)[:N]


def kernel(x, a, i, w0_1, w1_1, b1, p, w0_2, w1_2, b2, w0_3, w1_3, b3, wd, bd):
    del i
    pn = (p / jnp.sqrt(jnp.sum(p * p)))[:, None]

    a16, deg0 = _cast_deg(a)
    di0 = _dinv(deg0)
    xw0 = _small_mm(x, w0_1, jnp.zeros((1, 32), jnp.float32))
    xb0 = _small_mm(x, w1_1, b1[None, :])
    ones_col = jnp.ones((N, 1), jnp.float32)
    x1f, y1, _ = _conv_pass(a16, di0 * xw0, di0, xb0, pn, ones_col)
    y1 = y1[:, 0]

    k1 = int(np.ceil(0.5 * N))
    m1 = _topk_mask(y1, k1, jnp.ones((N,), jnp.float32))
    g1 = jax.nn.sigmoid(y1) * m1
    x1g = x1f * g1[:, None]

    deg1 = _big_mm(a16, m1[:, None])
    di1 = _dinv(deg1)
    xw1 = _small_mm(x1g, w0_2, jnp.zeros((1, 32), jnp.float32))
    xb1 = _small_mm(x1g, w1_2, b2[None, :])
    x2f, y2, _ = _conv_pass(a16, m1[:, None] * di1 * xw1, di1, xb1, pn,
                            m1[:, None])
    y2 = y2[:, 0]

    k2 = int(np.ceil(0.5 * k1))
    m2 = _topk_mask(y2, k2, m1)
    g2 = jax.nn.sigmoid(y2) * m2
    x2g = x2f * g2[:, None]

    deg2 = _big_mm(a16, m2[:, None])
    di2 = _dinv(deg2)
    xw2 = _small_mm(x2g, w0_3, jnp.zeros((1, 32), jnp.float32))
    xb2 = _small_mm(x2g, w1_3, b3[None, :])
    _, _, cs = _conv_pass(a16, m2[:, None] * di2 * xw2, di2, xb2, pn,
                          m2[:, None])

    pooled = jnp.sum(cs, axis=0) / k2
    return pooled @ wd + bd[None, :]


# speedup vs baseline: 3.7257x; 1.0779x over previous
import functools

import jax
import jax.numpy as jnp
import numpy as np
from jax.experimental import pallas as pl
from jax.experimental.pallas import tpu as pltpu

N = 10000
BR = 200


def _dinv(deg):
    return jnp.where(deg > 0, jax.lax.rsqrt(deg), 0.0)


def _cast_deg_body(a_ref, a16_ref, di_ref):
    blk = a_ref[...]
    a16_ref[...] = blk.astype(jnp.bfloat16)
    di_ref[...] = _dinv(jnp.sum(blk, axis=1, keepdims=True))


def _cast_deg(a):
    return pl.pallas_call(
        _cast_deg_body,
        grid=(N // BR,),
        in_specs=[pl.BlockSpec((BR, N), lambda i: (i, 0))],
        out_specs=[pl.BlockSpec((BR, N), lambda i: (i, 0)),
                   pl.BlockSpec((BR, 1), lambda i: (i, 0))],
        out_shape=[jax.ShapeDtypeStruct((N, N), jnp.bfloat16),
                   jax.ShapeDtypeStruct((N, 1), jnp.float32)],
    )(a)


def _deg_body(a_ref, m_ref, o_ref):
    deg = jnp.dot(a_ref[...], m_ref[...], preferred_element_type=jnp.float32)
    o_ref[...] = _dinv(deg)


def _deg_pass(a16, m):
    return pl.pallas_call(
        _deg_body,
        grid=(N // BR,),
        in_specs=[
            pl.BlockSpec((BR, N), lambda i: (i, 0)),
            pl.BlockSpec((N, 1), lambda i: (0, 0)),
        ],
        out_specs=pl.BlockSpec((BR, 1), lambda i: (i, 0)),
        out_shape=jax.ShapeDtypeStruct((N, 1), jnp.float32),
    )(a16, m.astype(a16.dtype))


def _conv1_body(a_ref, dif_ref, dib_ref, xw_ref, xb_ref, p_ref,
                o_feat, o_y, o_cs, v_scr):
    @pl.when(pl.program_id(0) == 0)
    def _():
        v_scr[...] = (dif_ref[...] * xw_ref[...]).astype(jnp.bfloat16)

    z = jnp.dot(a_ref[...], v_scr[...], preferred_element_type=jnp.float32)
    feat = jax.nn.relu(dib_ref[...] * z + xb_ref[...])
    o_feat[...] = feat
    o_y[...] = jnp.dot(feat, p_ref[...], preferred_element_type=jnp.float32)
    o_cs[...] = jnp.sum(feat, axis=0, keepdims=True)[None]


def _conv1(a16, di, xw, xb, p_col):
    h = xw.shape[1]
    return pl.pallas_call(
        _conv1_body,
        grid=(N // BR,),
        in_specs=[
            pl.BlockSpec((BR, N), lambda i: (i, 0)),
            pl.BlockSpec((N, 1), lambda i: (0, 0)),
            pl.BlockSpec((BR, 1), lambda i: (i, 0)),
            pl.BlockSpec((N, h), lambda i: (0, 0)),
            pl.BlockSpec((BR, h), lambda i: (i, 0)),
            pl.BlockSpec((h, 1), lambda i: (0, 0)),
        ],
        out_specs=[
            pl.BlockSpec((BR, h), lambda i: (i, 0)),
            pl.BlockSpec((BR, 1), lambda i: (i, 0)),
            pl.BlockSpec((1, 1, h), lambda i: (i, 0, 0)),
        ],
        out_shape=[
            jax.ShapeDtypeStruct((N, h), jnp.float32),
            jax.ShapeDtypeStruct((N, 1), jnp.float32),
            jax.ShapeDtypeStruct((N // BR, 1, h), jnp.float32),
        ],
        scratch_shapes=[pltpu.VMEM((N, h), jnp.bfloat16)],
    )(a16, di, di, xw, xb, p_col)


def _conv23_body(a_ref, dif_ref, dib_ref, y_ref, mf_ref, mb_ref, xp_ref,
                 w0_ref, w1_ref, b_ref, p_ref,
                 o_feat, o_y, o_cs, v_scr, xb_scr):
    i = pl.program_id(0)

    @pl.when(i == 0)
    def _():
        y = y_ref[...]
        g = mf_ref[...] / (1.0 + jnp.exp(-y))
        xg = xp_ref[...] * g
        xw = jnp.dot(xg, w0_ref[...], preferred_element_type=jnp.float32)
        v_scr[...] = (dif_ref[...] * xw).astype(jnp.bfloat16)
        xb_scr[...] = (jnp.dot(xg, w1_ref[...],
                               preferred_element_type=jnp.float32)
                       + b_ref[...])

    z = jnp.dot(a_ref[...], v_scr[...], preferred_element_type=jnp.float32)
    feat = (jax.nn.relu(dib_ref[...] * z + xb_scr[pl.ds(i * BR, BR), :])
            * mb_ref[...])
    o_feat[...] = feat
    o_y[...] = jnp.dot(feat, p_ref[...], preferred_element_type=jnp.float32)
    o_cs[...] = jnp.sum(feat, axis=0, keepdims=True)[None]


def _conv23(a16, di, y, m, xprev, w0, w1, b, p_col):
    h = xprev.shape[1]
    return pl.pallas_call(
        _conv23_body,
        grid=(N // BR,),
        in_specs=[
            pl.BlockSpec((BR, N), lambda i: (i, 0)),
            pl.BlockSpec((N, 1), lambda i: (0, 0)),
            pl.BlockSpec((BR, 1), lambda i: (i, 0)),
            pl.BlockSpec((N, 1), lambda i: (0, 0)),
            pl.BlockSpec((N, 1), lambda i: (0, 0)),
            pl.BlockSpec((BR, 1), lambda i: (i, 0)),
            pl.BlockSpec((N, h), lambda i: (0, 0)),
            pl.BlockSpec((h, h), lambda i: (0, 0)),
            pl.BlockSpec((h, h), lambda i: (0, 0)),
            pl.BlockSpec((1, h), lambda i: (0, 0)),
            pl.BlockSpec((h, 1), lambda i: (0, 0)),
        ],
        out_specs=[
            pl.BlockSpec((BR, h), lambda i: (i, 0)),
            pl.BlockSpec((BR, 1), lambda i: (i, 0)),
            pl.BlockSpec((1, 1, h), lambda i: (i, 0, 0)),
        ],
        out_shape=[
            jax.ShapeDtypeStruct((N, h), jnp.float32),
            jax.ShapeDtypeStruct((N, 1), jnp.float32),
            jax.ShapeDtypeStruct((N // BR, 1, h), jnp.float32),
        ],
        scratch_shapes=[pltpu.VMEM((N, h), jnp.bfloat16),
                        pltpu.VMEM((N, h), jnp.float32)],
    )(a16, di, di, y, m, m, xprev, w0, w1, b[None, :], p_col)


def _small_mm_body(x_ref, w_ref, b_ref, o_ref):
    o_ref[...] = (jnp.dot(x_ref[...], w_ref[...],
                          preferred_element_type=jnp.float32) + b_ref[...])


def _small_mm(x, w, b):
    n, f = x.shape
    h = w.shape[1]
    return pl.pallas_call(
        _small_mm_body,
        in_specs=[pl.BlockSpec((n, f), lambda: (0, 0)),
                  pl.BlockSpec((f, h), lambda: (0, 0)),
                  pl.BlockSpec((1, h), lambda: (0, 0))],
        out_specs=pl.BlockSpec((n, h), lambda: (0, 0)),
        out_shape=jax.ShapeDtypeStruct((n, h), jnp.float32),
    )(x, w, b)


NP_ROWS = 80
NP_COLS = 128


def _thresh_body(k, y_ref, valid_ref, o_ref):
    yi = jax.lax.bitcast_convert_type(y_ref[...], jnp.int32)
    key = yi ^ jnp.where(yi < 0, jnp.int32(0x7FFFFFFF), jnp.int32(0))
    u = jax.lax.bitcast_convert_type(key ^ jnp.int32(-0x80000000), jnp.uint32)
    u = jnp.where(valid_ref[...] > 0, u, jnp.uint32(0))

    def body(b, t):
        cand = t | (jnp.uint32(1) << (jnp.uint32(31) - b.astype(jnp.uint32)))
        cnt = jnp.sum((u >= cand).astype(jnp.int32))
        return jnp.where(cnt >= k, cand, t)

    t = jax.lax.fori_loop(0, 32, body, jnp.uint32(0))
    o_ref[...] = (u >= t).astype(jnp.float32)


def _topk_mask(y, k, valid):
    pad = NP_ROWS * NP_COLS - N
    y2 = jnp.pad(y, (0, pad), constant_values=-jnp.inf).reshape(NP_ROWS, NP_COLS)
    v2 = jnp.pad(valid, (0, pad)).reshape(NP_ROWS, NP_COLS)
    m = pl.pallas_call(
        functools.partial(_thresh_body, k),
        in_specs=[pl.BlockSpec((NP_ROWS, NP_COLS), lambda: (0, 0)),
                  pl.BlockSpec((NP_ROWS, NP_COLS), lambda: (0, 0))],
        out_specs=pl.BlockSpec((NP_ROWS, NP_COLS), lambda: (0, 0)),
        out_shape=jax.ShapeDtypeStruct((NP_ROWS, NP_COLS), jnp.float32),
    )(y2, v2)
    return m.reshape(-1)[:N]


def kernel(x, a, i, w0_1, w1_1, b1, p, w0_2, w1_2, b2, w0_3, w1_3, b3, wd, bd):
    del i
    pn = (p / jnp.sqrt(jnp.sum(p * p)))[:, None]

    a16, di0 = _cast_deg(a)
    xw0 = _small_mm(x, w0_1, jnp.zeros((1, 32), jnp.float32))
    xb0 = _small_mm(x, w1_1, b1[None, :])
    x1f, y1, _ = _conv1(a16, di0, xw0, xb0, pn)

    k1 = int(np.ceil(0.5 * N))
    m1 = _topk_mask(y1[:, 0], k1, jnp.ones((N,), jnp.float32))
    di1 = _deg_pass(a16, m1[:, None])
    x2f, y2, _ = _conv23(a16, di1, y1, m1[:, None], x1f, w0_2, w1_2, b2, pn)

    k2 = int(np.ceil(0.5 * k1))
    m2 = _topk_mask(y2[:, 0], k2, m1)
    di2 = _deg_pass(a16, m2[:, None])
    _, _, cs = _conv23(a16, di2, y2, m2[:, None], x2f, w0_3, w1_3, b3, pn)

    pooled = jnp.sum(cs, axis=0) / k2
    return pooled @ wd + bd[None, :]


# speedup vs baseline: 4.1721x; 1.1198x over previous
import functools

import jax
import jax.numpy as jnp
import numpy as np
from jax.experimental import pallas as pl
from jax.experimental.pallas import tpu as pltpu

N = 10000
BR = 400


def _dinv(deg):
    return jnp.where(deg > 0, jax.lax.rsqrt(deg), 0.0)


def _cast_deg_body(a_ref, a16_ref, di_ref):
    blk = a_ref[...]
    a16_ref[...] = blk.astype(jnp.bfloat16)
    di_ref[...] = _dinv(jnp.sum(blk, axis=1, keepdims=True))


def _cast_deg(a):
    return pl.pallas_call(
        _cast_deg_body,
        grid=(N // BR,),
        in_specs=[pl.BlockSpec((BR, N), lambda i: (i, 0))],
        out_specs=[pl.BlockSpec((BR, N), lambda i: (i, 0)),
                   pl.BlockSpec((BR, 1), lambda i: (i, 0))],
        out_shape=[jax.ShapeDtypeStruct((N, N), jnp.bfloat16),
                   jax.ShapeDtypeStruct((N, 1), jnp.float32)],
    )(a)


def _deg_body(a_ref, m_ref, o_ref):
    deg = jnp.dot(a_ref[...], m_ref[...], preferred_element_type=jnp.float32)
    o_ref[...] = _dinv(deg)


def _deg_pass(a16, m):
    return pl.pallas_call(
        _deg_body,
        grid=(N // BR,),
        in_specs=[
            pl.BlockSpec((BR, N), lambda i: (i, 0)),
            pl.BlockSpec((N, 1), lambda i: (0, 0)),
        ],
        out_specs=pl.BlockSpec((BR, 1), lambda i: (i, 0)),
        out_shape=jax.ShapeDtypeStruct((N, 1), jnp.float32),
    )(a16, m.astype(a16.dtype))


def _conv1_body(a_ref, dif_ref, dib_ref, xw_ref, xb_ref, p_ref,
                o_feat, o_y, o_cs, v_scr):
    @pl.when(pl.program_id(0) == 0)
    def _():
        v_scr[...] = (dif_ref[...] * xw_ref[...]).astype(jnp.bfloat16)

    z = jnp.dot(a_ref[...], v_scr[...], preferred_element_type=jnp.float32)
    feat = jax.nn.relu(dib_ref[...] * z + xb_ref[...])
    o_feat[...] = feat
    o_y[...] = jnp.dot(feat, p_ref[...], preferred_element_type=jnp.float32)
    o_cs[...] = jnp.sum(feat, axis=0, keepdims=True)[None]


def _conv1(a16, di, xw, xb, p_col):
    h = xw.shape[1]
    return pl.pallas_call(
        _conv1_body,
        grid=(N // BR,),
        in_specs=[
            pl.BlockSpec((BR, N), lambda i: (i, 0)),
            pl.BlockSpec((N, 1), lambda i: (0, 0)),
            pl.BlockSpec((BR, 1), lambda i: (i, 0)),
            pl.BlockSpec((N, h), lambda i: (0, 0)),
            pl.BlockSpec((BR, h), lambda i: (i, 0)),
            pl.BlockSpec((h, 1), lambda i: (0, 0)),
        ],
        out_specs=[
            pl.BlockSpec((BR, h), lambda i: (i, 0)),
            pl.BlockSpec((BR, 1), lambda i: (i, 0)),
            pl.BlockSpec((1, 1, h), lambda i: (i, 0, 0)),
        ],
        out_shape=[
            jax.ShapeDtypeStruct((N, h), jnp.float32),
            jax.ShapeDtypeStruct((N, 1), jnp.float32),
            jax.ShapeDtypeStruct((N // BR, 1, h), jnp.float32),
        ],
        scratch_shapes=[pltpu.VMEM((N, h), jnp.bfloat16)],
    )(a16, di, di, xw, xb, p_col)


def _conv23_body(a_ref, dif_ref, dib_ref, y_ref, mf_ref, mb_ref, xp_ref,
                 w0_ref, w1_ref, b_ref, p_ref,
                 o_feat, o_y, o_cs, v_scr, xb_scr):
    i = pl.program_id(0)

    @pl.when(i == 0)
    def _():
        y = y_ref[...]
        g = mf_ref[...] / (1.0 + jnp.exp(-y))
        xg = xp_ref[...] * g
        xw = jnp.dot(xg, w0_ref[...], preferred_element_type=jnp.float32)
        v_scr[...] = (dif_ref[...] * xw).astype(jnp.bfloat16)
        xb_scr[...] = (jnp.dot(xg, w1_ref[...],
                               preferred_element_type=jnp.float32)
                       + b_ref[...])

    z = jnp.dot(a_ref[...], v_scr[...], preferred_element_type=jnp.float32)
    feat = (jax.nn.relu(dib_ref[...] * z + xb_scr[pl.ds(i * BR, BR), :])
            * mb_ref[...])
    o_feat[...] = feat
    o_y[...] = jnp.dot(feat, p_ref[...], preferred_element_type=jnp.float32)
    o_cs[...] = jnp.sum(feat, axis=0, keepdims=True)[None]


def _conv23(a16, di, y, m, xprev, w0, w1, b, p_col):
    h = xprev.shape[1]
    return pl.pallas_call(
        _conv23_body,
        grid=(N // BR,),
        in_specs=[
            pl.BlockSpec((BR, N), lambda i: (i, 0)),
            pl.BlockSpec((N, 1), lambda i: (0, 0)),
            pl.BlockSpec((BR, 1), lambda i: (i, 0)),
            pl.BlockSpec((N, 1), lambda i: (0, 0)),
            pl.BlockSpec((N, 1), lambda i: (0, 0)),
            pl.BlockSpec((BR, 1), lambda i: (i, 0)),
            pl.BlockSpec((N, h), lambda i: (0, 0)),
            pl.BlockSpec((h, h), lambda i: (0, 0)),
            pl.BlockSpec((h, h), lambda i: (0, 0)),
            pl.BlockSpec((1, h), lambda i: (0, 0)),
            pl.BlockSpec((h, 1), lambda i: (0, 0)),
        ],
        out_specs=[
            pl.BlockSpec((BR, h), lambda i: (i, 0)),
            pl.BlockSpec((BR, 1), lambda i: (i, 0)),
            pl.BlockSpec((1, 1, h), lambda i: (i, 0, 0)),
        ],
        out_shape=[
            jax.ShapeDtypeStruct((N, h), jnp.float32),
            jax.ShapeDtypeStruct((N, 1), jnp.float32),
            jax.ShapeDtypeStruct((N // BR, 1, h), jnp.float32),
        ],
        scratch_shapes=[pltpu.VMEM((N, h), jnp.bfloat16),
                        pltpu.VMEM((N, h), jnp.float32)],
    )(a16, di, di, y, m, m, xprev, w0, w1, b[None, :], p_col)


def _small_mm_body(x_ref, w_ref, b_ref, o_ref):
    o_ref[...] = (jnp.dot(x_ref[...], w_ref[...],
                          preferred_element_type=jnp.float32) + b_ref[...])


def _small_mm(x, w, b):
    n, f = x.shape
    h = w.shape[1]
    return pl.pallas_call(
        _small_mm_body,
        in_specs=[pl.BlockSpec((n, f), lambda: (0, 0)),
                  pl.BlockSpec((f, h), lambda: (0, 0)),
                  pl.BlockSpec((1, h), lambda: (0, 0))],
        out_specs=pl.BlockSpec((n, h), lambda: (0, 0)),
        out_shape=jax.ShapeDtypeStruct((n, h), jnp.float32),
    )(x, w, b)


NP_ROWS = 80
NP_COLS = 128


def _thresh_body(k, y_ref, valid_ref, o_ref):
    yi = jax.lax.bitcast_convert_type(y_ref[...], jnp.int32)
    key = yi ^ jnp.where(yi < 0, jnp.int32(0x7FFFFFFF), jnp.int32(0))
    u = jax.lax.bitcast_convert_type(key ^ jnp.int32(-0x80000000), jnp.uint32)
    u = jnp.where(valid_ref[...] > 0, u, jnp.uint32(0))

    def body(b, t):
        cand = t | (jnp.uint32(1) << (jnp.uint32(31) - b.astype(jnp.uint32)))
        cnt = jnp.sum((u >= cand).astype(jnp.int32))
        return jnp.where(cnt >= k, cand, t)

    t = jax.lax.fori_loop(0, 32, body, jnp.uint32(0))
    o_ref[...] = (u >= t).astype(jnp.float32)


def _topk_mask(y, k, valid):
    pad = NP_ROWS * NP_COLS - N
    y2 = jnp.pad(y, (0, pad), constant_values=-jnp.inf).reshape(NP_ROWS, NP_COLS)
    v2 = jnp.pad(valid, (0, pad)).reshape(NP_ROWS, NP_COLS)
    m = pl.pallas_call(
        functools.partial(_thresh_body, k),
        in_specs=[pl.BlockSpec((NP_ROWS, NP_COLS), lambda: (0, 0)),
                  pl.BlockSpec((NP_ROWS, NP_COLS), lambda: (0, 0))],
        out_specs=pl.BlockSpec((NP_ROWS, NP_COLS), lambda: (0, 0)),
        out_shape=jax.ShapeDtypeStruct((NP_ROWS, NP_COLS), jnp.float32),
    )(y2, v2)
    return m.reshape(-1)[:N]


def kernel(x, a, i, w0_1, w1_1, b1, p, w0_2, w1_2, b2, w0_3, w1_3, b3, wd, bd):
    del i
    pn = (p / jnp.sqrt(jnp.sum(p * p)))[:, None]

    a16, di0 = _cast_deg(a)
    xw0 = _small_mm(x, w0_1, jnp.zeros((1, 32), jnp.float32))
    xb0 = _small_mm(x, w1_1, b1[None, :])
    x1f, y1, _ = _conv1(a16, di0, xw0, xb0, pn)

    k1 = int(np.ceil(0.5 * N))
    m1 = _topk_mask(y1[:, 0], k1, jnp.ones((N,), jnp.float32))
    di1 = _deg_pass(a16, m1[:, None])
    x2f, y2, _ = _conv23(a16, di1, y1, m1[:, None], x1f, w0_2, w1_2, b2, pn)

    k2 = int(np.ceil(0.5 * k1))
    m2 = _topk_mask(y2[:, 0], k2, m1)
    di2 = _deg_pass(a16, m2[:, None])
    _, _, cs = _conv23(a16, di2, y2, m2[:, None], x2f, w0_3, w1_3, b3, pn)

    pooled = jnp.sum(cs, axis=0) / k2
    return pooled @ wd + bd[None, :]


# speedup vs baseline: 4.2073x; 1.0085x over previous
import functools

import jax
import jax.numpy as jnp
import numpy as np
from jax.experimental import pallas as pl
from jax.experimental.pallas import tpu as pltpu

N = 10000
BR = 400
BRC = 400


def _dinv(deg):
    return jnp.where(deg > 0, jax.lax.rsqrt(deg), 0.0)


def _cast_deg_body(a_ref, a16_ref, di_ref):
    blk = a_ref[...]
    a16_ref[...] = blk.astype(jnp.bfloat16)
    di_ref[...] = _dinv(jnp.sum(blk, axis=1, keepdims=True))


def _cast_deg(a):
    return pl.pallas_call(
        _cast_deg_body,
        grid=(N // BRC,),
        in_specs=[pl.BlockSpec((BRC, N), lambda i: (i, 0))],
        out_specs=[pl.BlockSpec((BRC, N), lambda i: (i, 0)),
                   pl.BlockSpec((BRC, 1), lambda i: (i, 0))],
        out_shape=[jax.ShapeDtypeStruct((N, N), jnp.bfloat16),
                   jax.ShapeDtypeStruct((N, 1), jnp.float32)],
    )(a)


def _deg_body(a_ref, m_ref, o_ref):
    deg = jnp.dot(a_ref[...], m_ref[...], preferred_element_type=jnp.float32)
    o_ref[...] = _dinv(deg)


def _deg_pass(a16, m):
    return pl.pallas_call(
        _deg_body,
        grid=(N // BR,),
        in_specs=[
            pl.BlockSpec((BR, N), lambda i: (i, 0)),
            pl.BlockSpec((N, 1), lambda i: (0, 0)),
        ],
        out_specs=pl.BlockSpec((BR, 1), lambda i: (i, 0)),
        out_shape=jax.ShapeDtypeStruct((N, 1), jnp.float32),
    )(a16, m.astype(a16.dtype))


def _conv1_body(a_ref, dif_ref, dib_ref, xw_ref, xb_ref, p_ref,
                o_feat, o_y, o_cs, v_scr):
    @pl.when(pl.program_id(0) == 0)
    def _():
        v_scr[...] = (dif_ref[...] * xw_ref[...]).astype(jnp.bfloat16)

    z = jnp.dot(a_ref[...], v_scr[...], preferred_element_type=jnp.float32)
    feat = jax.nn.relu(dib_ref[...] * z + xb_ref[...])
    o_feat[...] = feat
    o_y[...] = jnp.dot(feat, p_ref[...], preferred_element_type=jnp.float32)
    o_cs[...] = jnp.sum(feat, axis=0, keepdims=True)[None]


def _conv1(a16, di, xw, xb, p_col):
    h = xw.shape[1]
    return pl.pallas_call(
        _conv1_body,
        grid=(N // BR,),
        in_specs=[
            pl.BlockSpec((BR, N), lambda i: (i, 0)),
            pl.BlockSpec((N, 1), lambda i: (0, 0)),
            pl.BlockSpec((BR, 1), lambda i: (i, 0)),
            pl.BlockSpec((N, h), lambda i: (0, 0)),
            pl.BlockSpec((BR, h), lambda i: (i, 0)),
            pl.BlockSpec((h, 1), lambda i: (0, 0)),
        ],
        out_specs=[
            pl.BlockSpec((BR, h), lambda i: (i, 0)),
            pl.BlockSpec((BR, 1), lambda i: (i, 0)),
            pl.BlockSpec((1, 1, h), lambda i: (i, 0, 0)),
        ],
        out_shape=[
            jax.ShapeDtypeStruct((N, h), jnp.float32),
            jax.ShapeDtypeStruct((N, 1), jnp.float32),
            jax.ShapeDtypeStruct((N // BR, 1, h), jnp.float32),
        ],
        scratch_shapes=[pltpu.VMEM((N, h), jnp.bfloat16)],
    )(a16, di, di, xw, xb, p_col)


def _conv23_body(a_ref, dif_ref, dib_ref, y_ref, mf_ref, mb_ref, xp_ref,
                 w0_ref, w1_ref, b_ref, p_ref,
                 o_feat, o_y, o_cs, v_scr, xb_scr):
    i = pl.program_id(0)

    @pl.when(i == 0)
    def _():
        y = y_ref[...]
        g = mf_ref[...] / (1.0 + jnp.exp(-y))
        xg = xp_ref[...] * g
        xw = jnp.dot(xg, w0_ref[...], preferred_element_type=jnp.float32)
        v_scr[...] = (dif_ref[...] * xw).astype(jnp.bfloat16)
        xb_scr[...] = (jnp.dot(xg, w1_ref[...],
                               preferred_element_type=jnp.float32)
                       + b_ref[...])

    z = jnp.dot(a_ref[...], v_scr[...], preferred_element_type=jnp.float32)
    feat = (jax.nn.relu(dib_ref[...] * z + xb_scr[pl.ds(i * BR, BR), :])
            * mb_ref[...])
    o_feat[...] = feat
    o_y[...] = jnp.dot(feat, p_ref[...], preferred_element_type=jnp.float32)
    o_cs[...] = jnp.sum(feat, axis=0, keepdims=True)[None]


def _conv23(a16, di, y, m, xprev, w0, w1, b, p_col):
    h = xprev.shape[1]
    return pl.pallas_call(
        _conv23_body,
        grid=(N // BR,),
        in_specs=[
            pl.BlockSpec((BR, N), lambda i: (i, 0)),
            pl.BlockSpec((N, 1), lambda i: (0, 0)),
            pl.BlockSpec((BR, 1), lambda i: (i, 0)),
            pl.BlockSpec((N, 1), lambda i: (0, 0)),
            pl.BlockSpec((N, 1), lambda i: (0, 0)),
            pl.BlockSpec((BR, 1), lambda i: (i, 0)),
            pl.BlockSpec((N, h), lambda i: (0, 0)),
            pl.BlockSpec((h, h), lambda i: (0, 0)),
            pl.BlockSpec((h, h), lambda i: (0, 0)),
            pl.BlockSpec((1, h), lambda i: (0, 0)),
            pl.BlockSpec((h, 1), lambda i: (0, 0)),
        ],
        out_specs=[
            pl.BlockSpec((BR, h), lambda i: (i, 0)),
            pl.BlockSpec((BR, 1), lambda i: (i, 0)),
            pl.BlockSpec((1, 1, h), lambda i: (i, 0, 0)),
        ],
        out_shape=[
            jax.ShapeDtypeStruct((N, h), jnp.float32),
            jax.ShapeDtypeStruct((N, 1), jnp.float32),
            jax.ShapeDtypeStruct((N // BR, 1, h), jnp.float32),
        ],
        scratch_shapes=[pltpu.VMEM((N, h), jnp.bfloat16),
                        pltpu.VMEM((N, h), jnp.float32)],
    )(a16, di, di, y, m, m, xprev, w0, w1, b[None, :], p_col)


def _proj_body(x_ref, w0_ref, w1_ref, b_ref, o0_ref, o1_ref):
    xb = x_ref[...]
    o0_ref[...] = jnp.dot(xb, w0_ref[...], preferred_element_type=jnp.float32)
    o1_ref[...] = (jnp.dot(xb, w1_ref[...],
                           preferred_element_type=jnp.float32) + b_ref[...])


def _proj(x, w0, w1, b):
    n, f = x.shape
    h = w0.shape[1]
    return pl.pallas_call(
        _proj_body,
        in_specs=[pl.BlockSpec((n, f), lambda: (0, 0)),
                  pl.BlockSpec((f, h), lambda: (0, 0)),
                  pl.BlockSpec((f, h), lambda: (0, 0)),
                  pl.BlockSpec((1, h), lambda: (0, 0))],
        out_specs=[pl.BlockSpec((n, h), lambda: (0, 0)),
                   pl.BlockSpec((n, h), lambda: (0, 0))],
        out_shape=[jax.ShapeDtypeStruct((n, h), jnp.float32),
                   jax.ShapeDtypeStruct((n, h), jnp.float32)],
    )(x, w0, w1, b)


NP_ROWS = 80
NP_COLS = 128


def _thresh_body(k, y_ref, valid_ref, o_ref):
    yi = jax.lax.bitcast_convert_type(y_ref[...], jnp.int32)
    key = yi ^ jnp.where(yi < 0, jnp.int32(0x7FFFFFFF), jnp.int32(0))
    u = jax.lax.bitcast_convert_type(key ^ jnp.int32(-0x80000000), jnp.uint32)
    u = jnp.where(valid_ref[...] > 0, u, jnp.uint32(0))

    def body(b, t):
        cand = t | (jnp.uint32(1) << (jnp.uint32(31) - b.astype(jnp.uint32)))
        cnt = jnp.sum((u >= cand).astype(jnp.int32))
        return jnp.where(cnt >= k, cand, t)

    t = jax.lax.fori_loop(0, 32, body, jnp.uint32(0))
    o_ref[...] = (u >= t).astype(jnp.float32)


def _topk_mask(y, k, valid):
    pad = NP_ROWS * NP_COLS - N
    y2 = jnp.pad(y, (0, pad), constant_values=-jnp.inf).reshape(NP_ROWS, NP_COLS)
    v2 = jnp.pad(valid, (0, pad)).reshape(NP_ROWS, NP_COLS)
    m = pl.pallas_call(
        functools.partial(_thresh_body, k),
        in_specs=[pl.BlockSpec((NP_ROWS, NP_COLS), lambda: (0, 0)),
                  pl.BlockSpec((NP_ROWS, NP_COLS), lambda: (0, 0))],
        out_specs=pl.BlockSpec((NP_ROWS, NP_COLS), lambda: (0, 0)),
        out_shape=jax.ShapeDtypeStruct((NP_ROWS, NP_COLS), jnp.float32),
    )(y2, v2)
    return m.reshape(-1)[:N]


def kernel(x, a, i, w0_1, w1_1, b1, p, w0_2, w1_2, b2, w0_3, w1_3, b3, wd, bd):
    del i
    pn = (p / jnp.sqrt(jnp.sum(p * p)))[:, None]

    a16, di0 = _cast_deg(a)
    xw0, xb0 = _proj(x, w0_1, w1_1, b1[None, :])
    x1f, y1, _ = _conv1(a16, di0, xw0, xb0, pn)

    k1 = int(np.ceil(0.5 * N))
    m1 = _topk_mask(y1[:, 0], k1, jnp.ones((N,), jnp.float32))
    di1 = _deg_pass(a16, m1[:, None])
    x2f, y2, _ = _conv23(a16, di1, y1, m1[:, None], x1f, w0_2, w1_2, b2, pn)

    k2 = int(np.ceil(0.5 * k1))
    m2 = _topk_mask(y2[:, 0], k2, m1)
    di2 = _deg_pass(a16, m2[:, None])
    _, _, cs = _conv23(a16, di2, y2, m2[:, None], x2f, w0_3, w1_3, b3, pn)

    pooled = jnp.sum(cs, axis=0) / k2
    return pooled @ wd + bd[None, :]
